# baseline (speedup 1.0000x reference)
import functools

import jax
import jax.numpy as jnp
from jax import lax
from jax.experimental import pallas as pl
from jax.experimental.pallas import tpu as pltpu
from jax.experimental.pallas import tpu_sc as plsc

NC = 2
NS = 16
L = 16

RG = 8
M_SC = 1536

TN = 128
MC = 256


def _nnd_tc_body(x_ref, y_ref, d1_ref, d2_ref, *, tn, mc):
    n = pl.program_id(1)
    x = x_ref[0]
    y = y_ref[0]
    m_total = y.shape[1]
    x0 = x[:, 0:1]
    x1 = x[:, 1:2]
    x2 = x[:, 2:3]
    rm = None
    for j in range(m_total // mc):
        ys = y[:, j * mc:(j + 1) * mc]
        d0 = x0 - ys[0:1, :]
        acc = d0 * d0
        d1 = x1 - ys[1:2, :]
        acc = acc + d1 * d1
        d2 = x2 - ys[2:3, :]
        acc = acc + d2 * d2
        rmj = jnp.min(acc, axis=1)
        rm = rmj if rm is None else jnp.minimum(rm, rmj)
        cmj = jnp.min(acc, axis=0)
        sl = pl.ds(j * mc, mc)
        prev = jnp.where(n == 0, jnp.full((mc,), jnp.inf, acc.dtype),
                         d2_ref[0, 0, sl])
        d2_ref[0, 0, sl] = jnp.minimum(prev, cmj)
    d1_ref[0, 0, pl.ds(n * tn, tn)] = rm


_DNUMS = lax.GatherDimensionNumbers(
    offset_dims=(), collapsed_slice_dims=(0,), start_index_map=(0,))


def _splat(v, lane):
    return lax.gather(v, jnp.full((L,), lane, jnp.int32)[:, None], _DNUMS,
                      slice_sizes=(1,),
                      mode=lax.GatherScatterMode.PROMISE_IN_BOUNDS)


def _nnd_sc_body(x_hbm, y_hbm, out1, out2,
                 x_ref, y_ref, cm_ref, rm_ref, tmp_ref, ob_ref, shared,
                 *, b_per_c, rows_w, m_tot, m_off):
    c = lax.axis_index("c")
    s = lax.axis_index("s")
    inf16 = jnp.full((L,), jnp.inf, jnp.float32)
    iota = lax.iota(jnp.int32, L)
    mchunks = m_tot // L
    cols_w = 128
    rw = m_tot // cols_w
    assert rw <= NS

    for bl in range(b_per_c):
        b = c * b_per_c + bl
        pltpu.sync_copy(y_hbm.at[b, :, pl.ds(m_off, m_tot)], y_ref)
        pltpu.sync_copy(x_hbm.at[b, s], x_ref)

        @plsc.parallel_loop(0, mchunks)
        def _init(i):
            cm_ref[pl.ds(i * L, L)] = inf16

        def group_body(g, carry):
            r0 = g * RG
            cb = (r0 // L) * L
            xv = [x_ref[pl.ds(d * rows_w + cb, L)] for d in range(3)]
            bc = [[_splat(xv[d], r0 % L + r) for d in range(3)]
                  for r in range(RG)]

            @plsc.parallel_loop(0, mchunks, carry=(inf16,) * RG, unroll=2)
            def rms(i, rms_c):
                off = i * L
                y0 = y_ref[0, pl.ds(off, L)]
                y1 = y_ref[1, pl.ds(off, L)]
                y2 = y_ref[2, pl.ds(off, L)]
                out = []
                ts = []
                for r in range(RG):
                    d0 = y0 - bc[r][0]
                    t = d0 * d0
                    d1 = y1 - bc[r][1]
                    t = t + d1 * d1
                    d2 = y2 - bc[r][2]
                    t = t + d2 * d2
                    out.append(jnp.minimum(rms_c[r], t))
                    ts.append(t)
                while len(ts) > 1:
                    ts = [jnp.minimum(ts[2 * k], ts[2 * k + 1])
                          for k in range(len(ts) // 2)]
                cm_ref[pl.ds(off, L)] = jnp.minimum(cm_ref[pl.ds(off, L)],
                                                    ts[0])
                return tuple(out)

            rv = rm_ref[pl.ds(cb, L)]
            base_lane = r0 % L
            for r in range(RG):
                mn = rms[r]
                for sh in (8, 4, 2, 1):
                    idx = (iota + sh) & (L - 1)
                    rot = lax.gather(
                        mn, idx[:, None], _DNUMS, slice_sizes=(1,),
                        mode=lax.GatherScatterMode.PROMISE_IN_BOUNDS)
                    mn = jnp.minimum(mn, rot)
                rv = jnp.where(iota == base_lane + r, mn, rv)
            rm_ref[pl.ds(cb, L)] = rv
            return carry

        lax.fori_loop(0, rows_w // RG, group_body, 0)

        pltpu.sync_copy(rm_ref, out1.at[b, pl.ds(s * rows_w, rows_w)])

        pltpu.sync_copy(cm_ref, shared.at[s])
        plsc.subcore_barrier()

        @pl.when(s < rw)
        def _reduce():
            pltpu.sync_copy(shared.at[:, pl.ds(s * cols_w, cols_w)],
                            tmp_ref)

            def red_body(j, carry):
                acc = tmp_ref[0, pl.ds(j * L, L)]
                for i in range(1, NS):
                    acc = jnp.minimum(acc, tmp_ref[i, pl.ds(j * L, L)])
                ob_ref[pl.ds(j * L, L)] = acc
                return carry
            lax.fori_loop(0, cols_w // L, red_body, 0)

            pltpu.sync_copy(ob_ref, out2.at[b, pl.ds(s * cols_w, cols_w)])

        plsc.subcore_barrier()


@jax.jit
def kernel(input1, input2):
    b, n, _ = input1.shape
    m = input2.shape[1]
    m_tc = m - M_SC
    rows_w = n // NS
    b_per_c = b // NC

    yt = input2.transpose(0, 2, 1)

    d1t, d2t = pl.pallas_call(
        functools.partial(_nnd_tc_body, tn=TN, mc=MC),
        grid=(b, n // TN),
        in_specs=[
            pl.BlockSpec((1, TN, 3), lambda b_, n_: (b_, n_, 0)),
            pl.BlockSpec((1, 3, m_tc), lambda b_, n_: (b_, 0, 0)),
        ],
        out_specs=[
            pl.BlockSpec((1, 1, n), lambda b_, n_: (b_, 0, 0)),
            pl.BlockSpec((1, 1, m_tc), lambda b_, n_: (b_, 0, 0)),
        ],
        out_shape=[
            jax.ShapeDtypeStruct((b, 1, n), jnp.float32),
            jax.ShapeDtypeStruct((b, 1, m_tc), jnp.float32),
        ],
    )(input1, yt)

    xt = input1.transpose(0, 2, 1)
    xw = xt.reshape(b, 3, NS, rows_w).transpose(0, 2, 1, 3)
    xw = xw.reshape(b, NS, 3 * rows_w)
    mesh = plsc.VectorSubcoreMesh(core_axis_name="c", subcore_axis_name="s",
                                  num_cores=NC, num_subcores=NS)
    body = functools.partial(_nnd_sc_body, b_per_c=b_per_c,
                             rows_w=rows_w, m_tot=M_SC, m_off=m_tc)
    d1s, d2s = pl.kernel(
        body,
        out_type=[jax.ShapeDtypeStruct((b, n), jnp.float32),
                  jax.ShapeDtypeStruct((b, M_SC), jnp.float32)],
        mesh=mesh,
        scratch_types=[
            pltpu.VMEM((3 * rows_w,), jnp.float32),
            pltpu.VMEM((3, M_SC), jnp.float32),
            pltpu.VMEM((M_SC,), jnp.float32),
            pltpu.VMEM((rows_w,), jnp.float32),
            pltpu.VMEM((NS, 128), jnp.float32),
            pltpu.VMEM((128,), jnp.float32),
            pltpu.VMEM_SHARED((NS, M_SC), jnp.float32),
        ],
    )(xw, yt)

    d1 = jnp.minimum(d1t.reshape(b, n), d1s)
    d2 = jnp.concatenate([d2t.reshape(b, m_tc), d2s], axis=1)
    return d1, d2


# speedup vs baseline: 1.1502x; 1.1502x over previous
import functools

import jax
import jax.numpy as jnp
from jax import lax
from jax.experimental import pallas as pl
from jax.experimental.pallas import tpu as pltpu
from jax.experimental.pallas import tpu_sc as plsc

NC = 2
NS = 16
L = 16

RG = 8
M_SC = 1280

TN = 128
MC = 256


def _nnd_tc_body(x_ref, y_ref, d1_ref, d2_ref, *, tn, mc):
    n = pl.program_id(1)
    x = x_ref[0]
    y = y_ref[0]
    m_total = y.shape[1]
    x0 = x[:, 0:1]
    x1 = x[:, 1:2]
    x2 = x[:, 2:3]
    rm = None
    for j in range(m_total // mc):
        ys = y[:, j * mc:(j + 1) * mc]
        d0 = x0 - ys[0:1, :]
        acc = d0 * d0
        d1 = x1 - ys[1:2, :]
        acc = acc + d1 * d1
        d2 = x2 - ys[2:3, :]
        acc = acc + d2 * d2
        rmj = jnp.min(acc, axis=1)
        rm = rmj if rm is None else jnp.minimum(rm, rmj)
        cmj = jnp.min(acc, axis=0)
        sl = pl.ds(j * mc, mc)
        prev = jnp.where(n == 0, jnp.full((mc,), jnp.inf, acc.dtype),
                         d2_ref[0, 0, sl])
        d2_ref[0, 0, sl] = jnp.minimum(prev, cmj)
    d1_ref[0, 0, pl.ds(n * tn, tn)] = rm


_DNUMS = lax.GatherDimensionNumbers(
    offset_dims=(), collapsed_slice_dims=(0,), start_index_map=(0,))


def _splat(v, lane):
    return lax.gather(v, jnp.full((L,), lane, jnp.int32)[:, None], _DNUMS,
                      slice_sizes=(1,),
                      mode=lax.GatherScatterMode.PROMISE_IN_BOUNDS)


def _nnd_sc_body(x_hbm, y_hbm, out1, out2,
                 x_ref, y_ref, cm_ref, rm_ref, tmp_ref, ob_ref, shared,
                 *, b_per_c, rows_w, m_tot, m_off):
    c = lax.axis_index("c")
    s = lax.axis_index("s")
    inf16 = jnp.full((L,), jnp.inf, jnp.float32)
    iota = lax.iota(jnp.int32, L)
    mchunks = m_tot // L
    cols_w = 128
    rw = m_tot // cols_w
    assert rw <= NS

    for bl in range(b_per_c):
        b = c * b_per_c + bl
        pltpu.sync_copy(y_hbm.at[b, :, pl.ds(m_off, m_tot)], y_ref)
        pltpu.sync_copy(x_hbm.at[b, s], x_ref)

        @plsc.parallel_loop(0, mchunks)
        def _init(i):
            cm_ref[pl.ds(i * L, L)] = inf16

        def group_body(g, carry):
            r0 = g * RG
            cb = (r0 // L) * L
            xv = [x_ref[pl.ds(d * rows_w + cb, L)] for d in range(3)]
            bc = [[_splat(xv[d], r0 % L + r) for d in range(3)]
                  for r in range(RG)]

            @plsc.parallel_loop(0, mchunks, carry=(inf16,) * RG, unroll=2)
            def rms(i, rms_c):
                off = i * L
                y0 = y_ref[0, pl.ds(off, L)]
                y1 = y_ref[1, pl.ds(off, L)]
                y2 = y_ref[2, pl.ds(off, L)]
                out = []
                ts = []
                for r in range(RG):
                    d0 = y0 - bc[r][0]
                    t = d0 * d0
                    d1 = y1 - bc[r][1]
                    t = t + d1 * d1
                    d2 = y2 - bc[r][2]
                    t = t + d2 * d2
                    out.append(jnp.minimum(rms_c[r], t))
                    ts.append(t)
                while len(ts) > 1:
                    ts = [jnp.minimum(ts[2 * k], ts[2 * k + 1])
                          for k in range(len(ts) // 2)]
                cm_ref[pl.ds(off, L)] = jnp.minimum(cm_ref[pl.ds(off, L)],
                                                    ts[0])
                return tuple(out)

            rv = rm_ref[pl.ds(cb, L)]
            base_lane = r0 % L
            for r in range(RG):
                mn = rms[r]
                for sh in (8, 4, 2, 1):
                    idx = (iota + sh) & (L - 1)
                    rot = lax.gather(
                        mn, idx[:, None], _DNUMS, slice_sizes=(1,),
                        mode=lax.GatherScatterMode.PROMISE_IN_BOUNDS)
                    mn = jnp.minimum(mn, rot)
                rv = jnp.where(iota == base_lane + r, mn, rv)
            rm_ref[pl.ds(cb, L)] = rv
            return carry

        lax.fori_loop(0, rows_w // RG, group_body, 0)

        pltpu.sync_copy(rm_ref, out1.at[b, pl.ds(s * rows_w, rows_w)])

        pltpu.sync_copy(cm_ref, shared.at[s])
        plsc.subcore_barrier()

        @pl.when(s < rw)
        def _reduce():
            pltpu.sync_copy(shared.at[:, pl.ds(s * cols_w, cols_w)],
                            tmp_ref)

            def red_body(j, carry):
                acc = tmp_ref[0, pl.ds(j * L, L)]
                for i in range(1, NS):
                    acc = jnp.minimum(acc, tmp_ref[i, pl.ds(j * L, L)])
                ob_ref[pl.ds(j * L, L)] = acc
                return carry
            lax.fori_loop(0, cols_w // L, red_body, 0)

            pltpu.sync_copy(ob_ref, out2.at[b, pl.ds(s * cols_w, cols_w)])

        plsc.subcore_barrier()


@jax.jit
def kernel(input1, input2):
    b, n, _ = input1.shape
    m = input2.shape[1]
    m_tc = m - M_SC
    rows_w = n // NS
    b_per_c = b // NC

    yt = input2.transpose(0, 2, 1)

    d1t, d2t = pl.pallas_call(
        functools.partial(_nnd_tc_body, tn=TN, mc=MC),
        grid=(b, n // TN),
        in_specs=[
            pl.BlockSpec((1, TN, 3), lambda b_, n_: (b_, n_, 0)),
            pl.BlockSpec((1, 3, m_tc), lambda b_, n_: (b_, 0, 0)),
        ],
        out_specs=[
            pl.BlockSpec((1, 1, n), lambda b_, n_: (b_, 0, 0)),
            pl.BlockSpec((1, 1, m_tc), lambda b_, n_: (b_, 0, 0)),
        ],
        out_shape=[
            jax.ShapeDtypeStruct((b, 1, n), jnp.float32),
            jax.ShapeDtypeStruct((b, 1, m_tc), jnp.float32),
        ],
    )(input1, yt)

    xt = input1.transpose(0, 2, 1)
    xw = xt.reshape(b, 3, NS, rows_w).transpose(0, 2, 1, 3)
    xw = xw.reshape(b, NS, 3 * rows_w)
    mesh = plsc.VectorSubcoreMesh(core_axis_name="c", subcore_axis_name="s",
                                  num_cores=NC, num_subcores=NS)
    body = functools.partial(_nnd_sc_body, b_per_c=b_per_c,
                             rows_w=rows_w, m_tot=M_SC, m_off=m_tc)
    d1s, d2s = pl.kernel(
        body,
        out_type=[jax.ShapeDtypeStruct((b, n), jnp.float32),
                  jax.ShapeDtypeStruct((b, M_SC), jnp.float32)],
        mesh=mesh,
        scratch_types=[
            pltpu.VMEM((3 * rows_w,), jnp.float32),
            pltpu.VMEM((3, M_SC), jnp.float32),
            pltpu.VMEM((M_SC,), jnp.float32),
            pltpu.VMEM((rows_w,), jnp.float32),
            pltpu.VMEM((NS, 128), jnp.float32),
            pltpu.VMEM((128,), jnp.float32),
            pltpu.VMEM_SHARED((NS, M_SC), jnp.float32),
        ],
    )(xw, yt)

    d1 = jnp.minimum(d1t.reshape(b, n), d1s)
    d2 = jnp.concatenate([d2t.reshape(b, m_tc), d2s], axis=1)
    return d1, d2
